# baseline (speedup 1.0000x reference)
import functools
import jax
import jax.numpy as jnp
from jax import lax
from jax.experimental import pallas as pl
from jax.experimental.pallas import tpu as pltpu
from jax.experimental.pallas import tpu_sc as plsc

_INFO = plsc.get_sparse_core_info()
_NC = _INFO.num_cores
_NS = _INFO.num_subcores
_NW = _NC * _NS


def _gather_native_out(table, index):
    B, = index.shape
    V, D = table.shape
    assert D == 32 and B % (_NW * 128) == 0
    b_per_w = B // _NW
    C = 512
    BLK = C // 128
    n_chunks = b_per_w // C
    blocks_per_w = b_per_w // 128
    TOTB = B // 128
    NWORDS = BLK * 1024

    mesh = plsc.VectorSubcoreMesh(core_axis_name="c", subcore_axis_name="s")

    @functools.partial(
        pl.kernel,
        mesh=mesh,
        out_type=jax.ShapeDtypeStruct(((D // 8) * TOTB * 1024,), table.dtype),
        scratch_types=[
            pltpu.VMEM((b_per_w,), jnp.int32),
            [pltpu.VMEM((C, D), table.dtype) for _ in range(2)],
            [pltpu.VMEM(((D // 8) * NWORDS,), table.dtype) for _ in range(2)],
            [pltpu.SemaphoreType.DMA for _ in range(2)],
            [pltpu.SemaphoreType.DMA for _ in range(2)],
        ],
        compiler_params=pltpu.CompilerParams(
            use_tc_tiling_on_sc=False, needs_layout_passes=False
        ),
    )
    def k(table_hbm, idx_hbm, out_hbm, idx_v, rows, nat, g_sems, w_sems):
        wid = lax.axis_index("s") * _NC + lax.axis_index("c")
        base = wid * b_per_w
        blk0 = wid * blocks_per_w
        pltpu.sync_copy(idx_hbm.at[pl.ds(base, b_per_w)], idx_v)

        j = lax.iota(jnp.int32, 16)
        cvs = [
            (j // 8 + 2 * h) * NWORDS + (j % 8) * 128 for h in range(D // 16)
        ]

        def start_gather(c, p):
            pltpu.async_copy(
                table_hbm.at[idx_v.at[pl.ds(c * C, C)]], rows[p], g_sems[p]
            )

        def wait_gather(c, p):
            pltpu.make_async_copy(
                table_hbm.at[idx_v.at[pl.ds(c * C, C)]], rows[p], g_sems[p]
            ).wait()

        def start_write(c, p):
            for fb in range(D // 8):
                pltpu.async_copy(
                    nat[p].at[pl.ds(fb * NWORDS, NWORDS)],
                    out_hbm.at[
                        pl.ds(fb * TOTB * 1024 + (blk0 + c * BLK) * 1024, NWORDS)
                    ],
                    w_sems[p],
                )

        def wait_write(c, p):
            for fb in range(D // 8):
                pltpu.make_async_copy(
                    nat[p].at[pl.ds(fb * NWORDS, NWORDS)],
                    out_hbm.at[
                        pl.ds(fb * TOTB * 1024 + (blk0 + c * BLK) * 1024, NWORDS)
                    ],
                    w_sems[p],
                ).wait()

        def transpose(p):
            def blk_body(blk, _):
                rowbase = blk * 128
                natbase = blk * 1024

                def r_body(r4, _):
                    for dr in range(4):
                        r = r4 * 4 + dr
                        dst = natbase + r
                        for h in range(D // 16):
                            vals = rows[p][rowbase + r, pl.ds(h * 16, 16)]
                            plsc.store_scatter(nat[p], [cvs[h] + dst], vals)
                    return 0

                lax.fori_loop(0, 32, r_body, 0)
                return 0

            lax.fori_loop(0, BLK, blk_body, 0)

        def do_chunk(c, p):
            wait_gather(c, p)

            @pl.when(c >= 2)
            def _():
                wait_write(c - 2, p)

            transpose(p)

            @pl.when(c + 2 < n_chunks)
            def _():
                start_gather(c + 2, p)

            start_write(c, p)

        start_gather(0, 0)
        start_gather(1, 1)

        def chunk_loop(c, _):
            @pl.when(c % 2 == 0)
            def _():
                do_chunk(c, 0)

            @pl.when(c % 2 == 1)
            def _():
                do_chunk(c, 1)

            return 0

        lax.fori_loop(0, n_chunks, chunk_loop, 0)
        wait_write(n_chunks - 2, 0)
        wait_write(n_chunks - 1, 1)

    return k(table, index)


def kernel(input, index, _):
    B, = index.shape
    V, D = input.shape
    out4d = _gather_native_out(input, index).reshape(D // 8, B // 128, 8, 128)
    out = out4d.transpose(0, 2, 1, 3).reshape(D, B).T
    return (input, index, out)
